# baseline (speedup 1.0000x reference)
import functools
import jax
import jax.numpy as jnp
from jax import lax
from jax.experimental import pallas as pl
from jax.experimental.pallas import tpu as pltpu
from jax.experimental.pallas import tpu_sc as plsc

BATCH, SEQ, MODEL_DIM = 4, 4096, 1024
ROWS = BATCH * SEQ
NC, NS, L = 2, 16, 16
NW = NC * NS
RPW = ROWS // NW
NVEC = RPW // L
CHUNK = L

_mesh = plsc.VectorSubcoreMesh(core_axis_name="c", subcore_axis_name="s")


@functools.partial(
    pl.kernel, mesh=_mesh,
    compiler_params=pltpu.CompilerParams(needs_layout_passes=False),
    out_type=jax.ShapeDtypeStruct((ROWS, MODEL_DIM), jnp.float32),
    scratch_types=[
        pltpu.VMEM((RPW,), jnp.int32),
        pltpu.VMEM((RPW + L,), jnp.int32),
        pltpu.VMEM((RPW + L,), jnp.int32),
        pltpu.VMEM((RPW + L,), jnp.int32),
        pltpu.VMEM((CHUNK, MODEL_DIM), jnp.float32),
        pltpu.VMEM((CHUNK, MODEL_DIM), jnp.float32),
        pltpu.SemaphoreType.DMA,
        pltpu.SemaphoreType.DMA,
    ],
)
def _sc_masker(seqs_hbm, mask_hbm, embed_hbm, out_hbm,
               maskv, srcu, dstu, dstm, rowbuf, embedbuf,
               sem_g, sem_s):
    wid = lax.axis_index("s") * NC + lax.axis_index("c")
    base = wid * RPW

    pltpu.sync_copy(mask_hbm.at[pl.ds(base, RPW)], maskv)
    for i in range(CHUNK):
        pltpu.sync_copy(embed_hbm, embedbuf.at[i])

    def comp_body(i, carry):
        cu, cm = carry
        mv = maskv[pl.ds(i * L, L)]
        mm = mv != 0
        mu = jnp.logical_not(mm)
        vals = base + i * L + lax.iota(jnp.int32, L)
        plsc.store_compressed(srcu.at[pl.ds(cu, L)], vals, mask=mu)
        plsc.store_compressed(dstu.at[pl.ds(cu, L)], vals, mask=mu)
        plsc.store_compressed(dstm.at[pl.ds(cm, L)], vals, mask=mm)
        nu = jnp.max(plsc.all_reduce_population_count(mu))
        return (cu + nu, cm + (L - nu))

    cu, cm = lax.fori_loop(0, NVEC, comp_body,
                           (jnp.int32(0), jnp.int32(0)))

    u0 = srcu[pl.ds(0, L)][0]
    m0 = dstm[pl.ds(0, L)][0]
    srcu[pl.ds(cu, L)] = jnp.full((L,), u0, jnp.int32)
    dstu[pl.ds(cu, L)] = jnp.full((L,), u0, jnp.int32)
    dstm[pl.ds(cm, L)] = jnp.full((L,), m0, jnp.int32)

    def copy_body(j, _):
        idxs = srcu[pl.ds(j * CHUNK, CHUNK)]
        pltpu.async_copy(seqs_hbm.at[idxs], rowbuf, sem_g).wait()
        idxd = dstu[pl.ds(j * CHUNK, CHUNK)]
        pltpu.async_copy(rowbuf, out_hbm.at[idxd], sem_s).wait()
        return 0

    ncu = (cu + CHUNK - 1) // CHUNK
    lax.fori_loop(0, ncu, copy_body, 0)

    def fill_body(j, _):
        idxm = dstm[pl.ds(j * CHUNK, CHUNK)]
        pltpu.async_copy(embedbuf, out_hbm.at[idxm], sem_s).wait()
        return 0

    ncm = (cm + CHUNK - 1) // CHUNK
    lax.fori_loop(0, ncm, fill_body, 0)


def kernel(seqs, temporal_mask, temporal_mask_embed):
    seqs2 = seqs.reshape(ROWS, MODEL_DIM)
    mask_i = temporal_mask.reshape(ROWS).astype(jnp.int32)
    out = _sc_masker(seqs2, mask_i, temporal_mask_embed)
    return (out.reshape(BATCH, SEQ, MODEL_DIM), temporal_mask)


# speedup vs baseline: 1.1079x; 1.1079x over previous
import functools
import jax
import jax.numpy as jnp
from jax import lax
from jax.experimental import pallas as pl
from jax.experimental.pallas import tpu as pltpu
from jax.experimental.pallas import tpu_sc as plsc

BATCH, SEQ, MODEL_DIM = 4, 4096, 1024
ROWS = BATCH * SEQ
NC, NS, L = 2, 16, 16
NW = NC * NS
RPW = ROWS // NW
NVEC = RPW // L
CHUNK = L

_mesh = plsc.VectorSubcoreMesh(core_axis_name="c", subcore_axis_name="s")


@functools.partial(
    pl.kernel, mesh=_mesh,
    compiler_params=pltpu.CompilerParams(needs_layout_passes=False),
    out_type=jax.ShapeDtypeStruct((ROWS, MODEL_DIM), jnp.float32),
    scratch_types=[
        pltpu.VMEM((RPW,), jnp.int32),
        pltpu.VMEM((RPW + L,), jnp.int32),
        pltpu.VMEM((RPW + L,), jnp.int32),
        pltpu.VMEM((CHUNK, MODEL_DIM), jnp.float32),
        pltpu.VMEM((CHUNK, MODEL_DIM), jnp.float32),
        pltpu.VMEM((CHUNK, MODEL_DIM), jnp.float32),
        pltpu.SemaphoreType.DMA,
        pltpu.SemaphoreType.DMA,
        pltpu.SemaphoreType.DMA,
    ],
)
def _sc_masker(seqs_hbm, mask_hbm, embed_hbm, out_hbm,
               maskv, idxu, idxm, rowbuf0, rowbuf1, embedbuf,
               sem_g, sem_s, sem_m):
    wid = lax.axis_index("s") * NC + lax.axis_index("c")
    base = wid * RPW

    pltpu.sync_copy(mask_hbm.at[pl.ds(base, RPW)], maskv)
    for i in range(CHUNK):
        pltpu.sync_copy(embed_hbm, embedbuf.at[i])

    def comp_body(i, carry):
        cu, cm = carry
        mv = maskv[pl.ds(i * L, L)]
        mm = mv != 0
        mu = jnp.logical_not(mm)
        vals = base + i * L + lax.iota(jnp.int32, L)
        plsc.store_compressed(idxu.at[pl.ds(cu, L)], vals, mask=mu)
        plsc.store_compressed(idxm.at[pl.ds(cm, L)], vals, mask=mm)
        nu = jnp.max(plsc.all_reduce_population_count(mu))
        return (cu + nu, cm + (L - nu))

    cu, cm = lax.fori_loop(0, NVEC, comp_body,
                           (jnp.int32(0), jnp.int32(0)))

    u0 = idxu[pl.ds(0, L)][0]
    m0 = idxm[pl.ds(0, L)][0]
    idxu[pl.ds(cu, L)] = jnp.full((L,), u0, jnp.int32)
    idxm[pl.ds(cm, L)] = jnp.full((L,), m0, jnp.int32)

    ncu = (cu + CHUNK - 1) // CHUNK
    ncm = (cm + CHUNK - 1) // CHUNK

    def fire_body(j, _):
        iv = idxm[pl.ds(j * CHUNK, CHUNK)]
        pltpu.async_copy(embedbuf, out_hbm.at[iv], sem_m)
        return 0

    lax.fori_loop(0, ncm, fire_body, 0)

    @pl.when(ncu > 0)
    def _():
        iv0 = idxu[pl.ds(0, CHUNK)]
        pltpu.async_copy(seqs_hbm.at[iv0], rowbuf0, sem_g)

    def copy_pair(jj, _):
        for b, (buf, nbuf) in enumerate(((rowbuf0, rowbuf1),
                                         (rowbuf1, rowbuf0))):
            j = 2 * jj + b

            @pl.when(j < ncu)
            def _(buf=buf, nbuf=nbuf, j=j):
                pltpu.make_async_copy(seqs_hbm.at[pl.ds(0, CHUNK)],
                                      buf, sem_g).wait()

                @pl.when(j + 1 < ncu)
                def _():
                    ivn = idxu[pl.ds((j + 1) * CHUNK, CHUNK)]
                    pltpu.async_copy(seqs_hbm.at[ivn], nbuf, sem_g)

                iv = idxu[pl.ds(j * CHUNK, CHUNK)]
                pltpu.async_copy(buf, out_hbm.at[iv], sem_s).wait()
        return 0

    lax.fori_loop(0, (ncu + 1) // 2, copy_pair, 0)

    def drain_body(j, _):
        pltpu.make_async_copy(embedbuf, out_hbm.at[pl.ds(0, CHUNK)],
                              sem_m).wait()
        return 0

    lax.fori_loop(0, ncm, drain_body, 0)


def kernel(seqs, temporal_mask, temporal_mask_embed):
    seqs2 = seqs.reshape(ROWS, MODEL_DIM)
    mask_i = temporal_mask.reshape(ROWS).astype(jnp.int32)
    out = _sc_masker(seqs2, mask_i, temporal_mask_embed)
    return (out.reshape(BATCH, SEQ, MODEL_DIM), temporal_mask)
